# baseline (speedup 1.0000x reference)
import functools

import jax
import jax.numpy as jnp
from jax import lax
from jax.experimental import pallas as pl
from jax.experimental.pallas import tpu as pltpu
from jax.experimental.pallas import tpu_sc as plsc

_BUFFER_SIZE = 10
_NUM_VARIABLES = 5

_WRITE_ROW = 0
_READ_ROW = (0 + 1 - 1) % _BUFFER_SIZE

_MESH = plsc.VectorSubcoreMesh(core_axis_name="c", subcore_axis_name="s")


@functools.partial(
    pl.kernel,
    out_type=jax.ShapeDtypeStruct((_NUM_VARIABLES,), jnp.float32),
    mesh=_MESH,
    scratch_types=[pltpu.VMEM((_BUFFER_SIZE, _NUM_VARIABLES), jnp.float32)],
)
def _delay_buffer_update(factors_hbm, history_hbm, out_hbm, hist_v):
    is_lead = jnp.logical_and(
        lax.axis_index("c") == 0, lax.axis_index("s") == 0
    )

    @pl.when(is_lead)
    def _():
        pltpu.sync_copy(history_hbm, hist_v)
        pltpu.sync_copy(factors_hbm, hist_v.at[_WRITE_ROW])
        pltpu.sync_copy(hist_v.at[_READ_ROW], out_hbm)


def kernel(causal_factors, causal_history, delay_weights):
    del delay_weights
    return _delay_buffer_update(causal_factors, causal_history)


# speedup vs baseline: 1.0402x; 1.0402x over previous
import functools

import jax
import jax.numpy as jnp
from jax import lax
from jax.experimental import pallas as pl
from jax.experimental.pallas import tpu as pltpu
from jax.experimental.pallas import tpu_sc as plsc

_BUFFER_SIZE = 10
_NUM_VARIABLES = 5

_WRITE_ROW = 0
_READ_ROW = (0 + 1 - 1) % _BUFFER_SIZE

_MESH = plsc.VectorSubcoreMesh(core_axis_name="c", subcore_axis_name="s")


@functools.partial(
    pl.kernel,
    out_type=jax.ShapeDtypeStruct((_NUM_VARIABLES,), jnp.float32),
    mesh=_MESH,
    scratch_types=[pltpu.VMEM((_BUFFER_SIZE, _NUM_VARIABLES), jnp.float32)],
)
def _delay_buffer_update(factors_hbm, history_hbm, out_hbm, hist_v):
    is_lead = jnp.logical_and(
        lax.axis_index("c") == 0, lax.axis_index("s") == 0
    )

    @pl.when(is_lead)
    def _():
        pltpu.sync_copy(factors_hbm, hist_v.at[_WRITE_ROW])
        pltpu.sync_copy(hist_v.at[_READ_ROW], out_hbm)


def kernel(causal_factors, causal_history, delay_weights):
    del delay_weights
    return _delay_buffer_update(causal_factors, causal_history)


# speedup vs baseline: 1.1064x; 1.0637x over previous
import functools

import jax
import jax.numpy as jnp
from jax import lax
from jax.experimental import pallas as pl
from jax.experimental.pallas import tpu as pltpu
from jax.experimental.pallas import tpu_sc as plsc

_BUFFER_SIZE = 10
_NUM_VARIABLES = 5

_WRITE_ROW = 0
_READ_ROW = (0 + 1 - 1) % _BUFFER_SIZE

_MESH = plsc.ScalarSubcoreMesh(axis_name="c", num_cores=2)


@functools.partial(
    pl.kernel,
    out_type=jax.ShapeDtypeStruct((_NUM_VARIABLES,), jnp.float32),
    mesh=_MESH,
    scratch_types=[
        pltpu.VMEM_SHARED((_BUFFER_SIZE, _NUM_VARIABLES), jnp.float32)
    ],
)
def _delay_buffer_update(factors_hbm, history_hbm, out_hbm, hist_spmem):
    del history_hbm

    @pl.when(lax.axis_index("c") == 0)
    def _():
        pltpu.sync_copy(factors_hbm, hist_spmem.at[_WRITE_ROW])
        pltpu.sync_copy(hist_spmem.at[_READ_ROW], out_hbm)


def kernel(causal_factors, causal_history, delay_weights):
    del delay_weights
    return _delay_buffer_update(causal_factors, causal_history)


# speedup vs baseline: 1.2143x; 1.0975x over previous
import functools

import jax
import jax.numpy as jnp
from jax import lax
from jax.experimental import pallas as pl
from jax.experimental.pallas import tpu as pltpu
from jax.experimental.pallas import tpu_sc as plsc

_BUFFER_SIZE = 10
_NUM_VARIABLES = 5

_WRITE_ROW = 0
_READ_ROW = (0 + 1 - 1) % _BUFFER_SIZE

_MESH = plsc.ScalarSubcoreMesh(axis_name="c", num_cores=1)


@functools.partial(
    pl.kernel,
    out_type=jax.ShapeDtypeStruct((_NUM_VARIABLES,), jnp.float32),
    mesh=_MESH,
    scratch_types=[
        pltpu.VMEM_SHARED((_BUFFER_SIZE, _NUM_VARIABLES), jnp.float32)
    ],
)
def _delay_buffer_update(factors_hbm, out_hbm, hist_spmem):
    pltpu.sync_copy(factors_hbm, hist_spmem.at[_WRITE_ROW])
    pltpu.sync_copy(hist_spmem.at[_READ_ROW], out_hbm)


def kernel(causal_factors, causal_history, delay_weights):
    del causal_history, delay_weights
    return _delay_buffer_update(causal_factors)


# speedup vs baseline: 7.3536x; 6.0561x over previous
import jax
import jax.numpy as jnp
from jax.experimental import pallas as pl
from jax.experimental.pallas import tpu as pltpu

_BUFFER_SIZE = 10
_NUM_VARIABLES = 5
_WRITE_ROW = 0
_READ_ROW = 0


def _body(f_ref, h_ref, o_ref, s_ref):
    s_ref[...] = h_ref[...]
    s_ref[_WRITE_ROW, :] = f_ref[0, :]
    o_ref[0, :] = s_ref[_READ_ROW, :]


def kernel(causal_factors, causal_history, delay_weights):
    del delay_weights
    out = pl.pallas_call(
        _body,
        out_shape=jax.ShapeDtypeStruct((1, _NUM_VARIABLES), jnp.float32),
        scratch_shapes=[pltpu.VMEM((_BUFFER_SIZE, _NUM_VARIABLES), jnp.float32)],
    )(causal_factors.reshape(1, _NUM_VARIABLES), causal_history)
    return out.reshape(_NUM_VARIABLES)


# speedup vs baseline: 17.0987x; 2.3252x over previous
import jax
import jax.numpy as jnp
from jax.experimental import pallas as pl
from jax.experimental.pallas import tpu as pltpu

_BUFFER_SIZE = 10
_NUM_VARIABLES = 5
_WRITE_ROW = 0
_READ_ROW = 0


def _body(f_ref, o_ref, s_ref):
    s_ref[_WRITE_ROW, :] = f_ref[...]
    o_ref[...] = s_ref[_READ_ROW, :]


def kernel(causal_factors, causal_history, delay_weights):
    del causal_history, delay_weights
    return pl.pallas_call(
        _body,
        out_shape=jax.ShapeDtypeStruct((_NUM_VARIABLES,), jnp.float32),
        scratch_shapes=[pltpu.VMEM((_BUFFER_SIZE, _NUM_VARIABLES), jnp.float32)],
    )(causal_factors)
